# baseline (speedup 1.0000x reference)
import jax
import jax.numpy as jnp
from jax.experimental import pallas as pl
from jax.experimental.pallas import tpu as pltpu

LAYERS = 3
N = 321
NP = 336
NT = 56
TGRID = NP // NT
L = 96
L2 = L // 2
D = 64
DP = 2 * D
FF = 64
FF2 = 4 * FF
E = 4
B = 8
P = 96
LD = L * D
RT2 = NT * L2
F32 = jnp.float32
BF16 = jnp.bfloat16


def _routing_compute(logits, W1s, b1s, W2s, b2s):
    col = jax.lax.broadcasted_iota(jnp.int32, (B, E), 1)
    m1 = jnp.max(logits, axis=1, keepdims=True)
    i1 = jnp.min(jnp.where(logits == m1, col, E), axis=1, keepdims=True)
    masked = jnp.where(col == i1, -jnp.inf, logits)
    m2 = jnp.max(masked, axis=1, keepdims=True)
    i2 = jnp.min(jnp.where(masked == m2, col, E), axis=1, keepdims=True)
    e2 = jnp.exp(m2 - m1)
    denom = 1.0 + e2
    g1 = 1.0 / denom
    g2 = e2 / denom
    gates = jnp.where(col == i1, g1, 0.0) + jnp.where(col == i2, g2, 0.0)

    w1a = jnp.zeros((B, D, FF), F32)
    w1b = jnp.zeros((B, D, FF), F32)
    w2a = jnp.zeros((B, FF, D), F32)
    w2b = jnp.zeros((B, FF, D), F32)
    b1a = jnp.zeros((B, FF), F32)
    b1b = jnp.zeros((B, FF), F32)
    b2c = jnp.zeros((B, D), F32)
    for e in range(E):
        s1 = (i1 == e).astype(F32)
        s2 = (i2 == e).astype(F32)
        sg1 = g1 * s1
        sg2 = g2 * s2
        w1a = w1a + s1[:, :, None] * W1s[e][None]
        w1b = w1b + s2[:, :, None] * W1s[e][None]
        w2a = w2a + sg1[:, :, None] * W2s[e][None]
        w2b = w2b + sg2[:, :, None] * W2s[e][None]
        b1a = b1a + s1 * b1s[e][None, :]
        b1b = b1b + s2 * b1s[e][None, :]
        b2c = b2c + (sg1 + sg2) * b2s[e][None, :]
    w1cat = jnp.concatenate([w1a, w1b], axis=2)
    w2cat = jnp.concatenate([w2a, w2b], axis=1)
    b1cat = jnp.concatenate([b1a, b1b], axis=1)
    zw1 = jnp.zeros((B, D, 2 * FF), F32)
    w1p = jnp.concatenate([
        jnp.concatenate([w1cat, zw1], axis=2),
        jnp.concatenate([zw1, w1cat], axis=2)], axis=1)
    zw2 = jnp.zeros((B, 2 * FF, D), F32)
    w2p = jnp.concatenate([
        jnp.concatenate([w2cat, zw2], axis=2),
        jnp.concatenate([zw2, w2cat], axis=2)], axis=1)
    b1p = jnp.concatenate([b1cat, b1cat], axis=1)
    b2p = jnp.concatenate([b2c, b2c], axis=1)
    return w1p, b1p[:, None, :], w2p, b2p[:, None, :], gates


def _routing0_body(lg_ref, W1_ref, b1_ref, W2_ref, b2_ref,
                   w1_ref, bb1_ref, w2_ref, bb2_ref, g_ref):
    o = _routing_compute(lg_ref[...], W1_ref, b1_ref, W2_ref, b2_ref)
    w1_ref[...], bb1_ref[...], w2_ref[...], bb2_ref[...], g_ref[...] = o


def _routing_body(gi_ref, gw_ref, W1_ref, b1_ref, W2_ref, b2_ref,
                  w1_ref, bb1_ref, w2_ref, bb2_ref, g_ref):
    logits = jnp.dot(gi_ref[...], gw_ref[...],
                     preferred_element_type=F32) * (1.0 / (L * D))
    o = _routing_compute(logits, W1_ref, b1_ref, W2_ref, b2_ref)
    w1_ref[...], bb1_ref[...], w2_ref[...], bb2_ref[...], g_ref[...] = o


_ROUT_OUT = (
    jax.ShapeDtypeStruct((B, DP, FF2), F32),
    jax.ShapeDtypeStruct((B, 1, FF2), F32),
    jax.ShapeDtypeStruct((B, FF2, DP), F32),
    jax.ShapeDtypeStruct((B, 1, DP), F32),
    jax.ShapeDtypeStruct((B, E), F32),
)


def _routing0(logits0, W1s, b1s, W2s, b2s):
    return pl.pallas_call(_routing0_body, out_shape=_ROUT_OUT)(
        logits0, W1s, b1s, W2s, b2s)


def _routing(gi, gw, W1s, b1s, W2s, b2s):
    return pl.pallas_call(_routing_body, out_shape=_ROUT_OUT)(
        gi, gw, W1s, b1s, W2s, b2s)


def _ffn_y(Xb, w1_ref, b1_ref, w2_ref, b2_ref):
    h = jnp.dot(Xb, w1_ref[0], preferred_element_type=F32) + b1_ref[0]
    h = jnp.maximum(h, 0.0).astype(BF16)
    return jnp.dot(h, w2_ref[0], preferred_element_type=F32) + b2_ref[0]


def _node_sums(o):
    o3 = o.reshape(NT, L2, DP)
    return jnp.sum(jnp.sum(o3, axis=1), axis=1, keepdims=True)


def _layer0_body(x_ref, m_ref, s_ref, sw_ref, sb_ref,
                 w1_ref, b1_ref, w2_ref, b2_ref, out_ref, gate_ref):
    xn = (x_ref[0] - m_ref[0]) / s_ref[0]
    xe = xn[:, :L2]
    xo = xn[:, L2:]
    sw = sw_ref[...][None]
    sb = sb_ref[...][None]
    Xp = jnp.concatenate([xe[:, :, None] * sw + sb,
                          xo[:, :, None] * sw + sb], axis=2)
    X = Xp.reshape(RT2, DP)
    y = _ffn_y(X.astype(BF16), w1_ref, b1_ref, w2_ref, b2_ref)
    o = X + y
    out_ref[0] = o.astype(BF16)
    gate_ref[0, 0] = _node_sums(o)


def _layer_body(a_ref, gin_ref, w1_ref, b1_ref, w2_ref, b2_ref,
                out_ref, gate_ref):
    Xb = a_ref[0]
    y = _ffn_y(Xb, w1_ref, b1_ref, w2_ref, b2_ref)
    out_ref[0] = (Xb.astype(F32) + y).astype(BF16)
    gate_ref[0, 0] = gin_ref[0, 0] + _node_sums(y)


def _layer2_proj_body(a_ref, w1_ref, b1_ref, w2_ref, b2_ref,
                      pw_ref, pb_ref, m_ref, s_ref, o_ref):
    Xb = a_ref[0]
    y = _ffn_y(Xb, w1_ref, b1_ref, w2_ref, b2_ref)
    o3 = (Xb.astype(F32) + y).astype(BF16).reshape(NT, L2, DP)
    om = jnp.concatenate([o3[:, j, :] for j in range(L2)], axis=1)
    yp = jnp.dot(om, pw_ref[...], preferred_element_type=F32) + pb_ref[...]
    o_ref[0] = yp * s_ref[0] + m_ref[0]


_W_SPECS = [
    pl.BlockSpec((1, DP, FF2), lambda b, t: (b, 0, 0)),
    pl.BlockSpec((1, 1, FF2), lambda b, t: (b, 0, 0)),
    pl.BlockSpec((1, FF2, DP), lambda b, t: (b, 0, 0)),
    pl.BlockSpec((1, 1, DP), lambda b, t: (b, 0, 0)),
]
_A_SPEC = pl.BlockSpec((1, RT2, DP), lambda b, t: (b, t, 0))
_GATE_OUT_BF16 = (
    jax.ShapeDtypeStruct((B, NP * L2, DP), BF16),
    jax.ShapeDtypeStruct((B, TGRID, NT, 1), F32),
)
_GATE_OUT_SPECS = (
    _A_SPEC,
    pl.BlockSpec((1, 1, NT, 1), lambda b, t: (b, t, 0, 0)),
)


def _layer0(x_tp, m3, s3, sw, sb, w1c, b1c, w2c, b2c):
    return pl.pallas_call(
        _layer0_body,
        grid=(B, TGRID),
        in_specs=[
            pl.BlockSpec((1, NT, L), lambda b, t: (b, t, 0)),
            pl.BlockSpec((1, NT, 1), lambda b, t: (b, t, 0)),
            pl.BlockSpec((1, NT, 1), lambda b, t: (b, t, 0)),
            pl.BlockSpec((1, D), lambda b, t: (0, 0)),
            pl.BlockSpec((1, D), lambda b, t: (0, 0)),
            *_W_SPECS,
        ],
        out_specs=_GATE_OUT_SPECS,
        out_shape=_GATE_OUT_BF16,
    )(x_tp, m3, s3, sw, sb, w1c.astype(BF16), b1c, w2c.astype(BF16), b2c)


def _layer(A, gsum, w1c, b1c, w2c, b2c):
    return pl.pallas_call(
        _layer_body,
        grid=(B, TGRID),
        in_specs=[_A_SPEC,
                  pl.BlockSpec((1, 1, NT, 1), lambda b, t: (b, t, 0, 0)),
                  *_W_SPECS],
        out_specs=_GATE_OUT_SPECS,
        out_shape=_GATE_OUT_BF16,
    )(A, gsum, w1c.astype(BF16), b1c, w2c.astype(BF16), b2c)


def _layer2_proj(A, w1c, b1c, w2c, b2c, pw, pb, m3, s3):
    return pl.pallas_call(
        _layer2_proj_body,
        grid=(B, TGRID),
        in_specs=[
            _A_SPEC, *_W_SPECS,
            pl.BlockSpec((LD, P), lambda b, t: (0, 0)),
            pl.BlockSpec((1, P), lambda b, t: (0, 0)),
            pl.BlockSpec((1, NT, 1), lambda b, t: (b, t, 0)),
            pl.BlockSpec((1, NT, 1), lambda b, t: (b, t, 0)),
        ],
        out_specs=pl.BlockSpec((1, NT, P), lambda b, t: (b, t, 0)),
        out_shape=jax.ShapeDtypeStruct((B, NP, P), F32),
    )(A, w1c.astype(BF16), b1c, w2c.astype(BF16), b2c, pw, pb, m3, s3)


def _stats_body(g_ref, bal_ref, con_ref):
    g = g_ref[...]
    imp = jnp.sum(g, axis=1)
    mean = jnp.mean(imp, axis=1, keepdims=True)
    var = jnp.mean((imp - mean) ** 2, axis=1, keepdims=True)
    bal = var / (mean ** 2 + 1e-10)
    bal_ref[...] = jnp.sum(bal, axis=0, keepdims=True)
    con_l = -jnp.mean(jnp.sum(g * jnp.log(g + 1e-9), axis=2),
                      axis=1, keepdims=True)
    con_ref[...] = jnp.mean(con_l, axis=0, keepdims=True)


def _stats(gates_all):
    return pl.pallas_call(
        _stats_body,
        out_shape=(jax.ShapeDtypeStruct((1, 1), F32),
                   jax.ShapeDtypeStruct((1, 1), F32)),
    )(gates_all)


def kernel(x, start_w, start_b, gate_w, W1, b1, W2, b2, proj_w, proj_b):
    means = x.mean(axis=1, keepdims=True)
    std = jnp.sqrt(x.var(axis=1, keepdims=True) + 1e-5)
    xn = (x - means) / std
    out0 = xn[..., None] * start_w + start_b
    gate_in0 = out0.mean(axis=(1, 3))
    logits0 = gate_in0 @ gate_w[0]

    m3 = jnp.pad(means[:, 0, :], ((0, 0), (0, NP - N)))[:, :, None]
    s3 = jnp.pad(std[:, 0, :], ((0, 0), (0, NP - N)),
                 constant_values=1.0)[:, :, None]
    x_t = jnp.pad(x.transpose(0, 2, 1), ((0, 0), (0, NP - N), (0, 0)))
    x_tp = jnp.concatenate([x_t[:, :, 0::2], x_t[:, :, 1::2]], axis=2)
    gw_p = jnp.pad(gate_w, ((0, 0), (0, NP - N), (0, 0)))
    sw = start_w[None, :]
    sb = start_b[None, :]
    pb = proj_b[None, :]

    w1c, b1c, w2c, b2c, g0 = _routing0(logits0, W1[0], b1[0], W2[0], b2[0])
    A, gsum = _layer0(x_tp, m3, s3, sw, sb, w1c, b1c, w2c, b2c)
    w1c, b1c, w2c, b2c, g1 = _routing(gsum.reshape(B, NP), gw_p[1],
                                      W1[1], b1[1], W2[1], b2[1])
    A, gsum = _layer(A, gsum, w1c, b1c, w2c, b2c)
    w1c, b1c, w2c, b2c, g2 = _routing(gsum.reshape(B, NP), gw_p[2],
                                      W1[2], b1[2], W2[2], b2[2])
    o_nd = _layer2_proj(A, w1c, b1c, w2c, b2c, proj_w.astype(BF16),
                        pb, m3, s3)
    out = o_nd[:, :N, :].transpose(0, 2, 1)
    bal, con = _stats(jnp.stack([g0, g1, g2]))
    return out, bal[0, 0], con[0, 0]


# speedup vs baseline: 1.0807x; 1.0807x over previous
import jax
import jax.numpy as jnp
from jax.experimental import pallas as pl
from jax.experimental.pallas import tpu as pltpu

LAYERS = 3
N = 321
NP = 336
NT = 112
TGRID = NP // NT
L = 96
L2 = L // 2
D = 64
DP = 2 * D
FF = 64
FF2 = 4 * FF
E = 4
B = 8
P = 96
LD = L * D
RT2 = NT * L2
F32 = jnp.float32
BF16 = jnp.bfloat16


def _routing_compute(logits, W1s, b1s, W2s, b2s):
    col = jax.lax.broadcasted_iota(jnp.int32, (B, E), 1)
    m1 = jnp.max(logits, axis=1, keepdims=True)
    i1 = jnp.min(jnp.where(logits == m1, col, E), axis=1, keepdims=True)
    masked = jnp.where(col == i1, -jnp.inf, logits)
    m2 = jnp.max(masked, axis=1, keepdims=True)
    i2 = jnp.min(jnp.where(masked == m2, col, E), axis=1, keepdims=True)
    e2 = jnp.exp(m2 - m1)
    denom = 1.0 + e2
    g1 = 1.0 / denom
    g2 = e2 / denom
    gates = jnp.where(col == i1, g1, 0.0) + jnp.where(col == i2, g2, 0.0)

    w1a = jnp.zeros((B, D, FF), F32)
    w1b = jnp.zeros((B, D, FF), F32)
    w2a = jnp.zeros((B, FF, D), F32)
    w2b = jnp.zeros((B, FF, D), F32)
    b1a = jnp.zeros((B, FF), F32)
    b1b = jnp.zeros((B, FF), F32)
    b2c = jnp.zeros((B, D), F32)
    for e in range(E):
        s1 = (i1 == e).astype(F32)
        s2 = (i2 == e).astype(F32)
        sg1 = g1 * s1
        sg2 = g2 * s2
        w1a = w1a + s1[:, :, None] * W1s[e][None]
        w1b = w1b + s2[:, :, None] * W1s[e][None]
        w2a = w2a + sg1[:, :, None] * W2s[e][None]
        w2b = w2b + sg2[:, :, None] * W2s[e][None]
        b1a = b1a + s1 * b1s[e][None, :]
        b1b = b1b + s2 * b1s[e][None, :]
        b2c = b2c + (sg1 + sg2) * b2s[e][None, :]
    w1cat = jnp.concatenate([w1a, w1b], axis=2)
    w2cat = jnp.concatenate([w2a, w2b], axis=1)
    b1cat = jnp.concatenate([b1a, b1b], axis=1)
    zw1 = jnp.zeros((B, D, 2 * FF), F32)
    w1p = jnp.concatenate([
        jnp.concatenate([w1cat, zw1], axis=2),
        jnp.concatenate([zw1, w1cat], axis=2)], axis=1)
    zw2 = jnp.zeros((B, 2 * FF, D), F32)
    w2p = jnp.concatenate([
        jnp.concatenate([w2cat, zw2], axis=2),
        jnp.concatenate([zw2, w2cat], axis=2)], axis=1)
    b1p = jnp.concatenate([b1cat, b1cat], axis=1)
    b2p = jnp.concatenate([b2c, b2c], axis=1)
    return w1p, b1p[:, None, :], w2p, b2p[:, None, :], gates


def _routing0_body(lg_ref, W1_ref, b1_ref, W2_ref, b2_ref,
                   w1_ref, bb1_ref, w2_ref, bb2_ref, g_ref):
    o = _routing_compute(lg_ref[...], W1_ref, b1_ref, W2_ref, b2_ref)
    w1_ref[...], bb1_ref[...], w2_ref[...], bb2_ref[...], g_ref[...] = o


def _routing_body(gi_ref, gw_ref, W1_ref, b1_ref, W2_ref, b2_ref,
                  w1_ref, bb1_ref, w2_ref, bb2_ref, g_ref):
    logits = jnp.dot(gi_ref[...], gw_ref[...],
                     preferred_element_type=F32) * (1.0 / (L * D))
    o = _routing_compute(logits, W1_ref, b1_ref, W2_ref, b2_ref)
    w1_ref[...], bb1_ref[...], w2_ref[...], bb2_ref[...], g_ref[...] = o


_ROUT_OUT = (
    jax.ShapeDtypeStruct((B, DP, FF2), F32),
    jax.ShapeDtypeStruct((B, 1, FF2), F32),
    jax.ShapeDtypeStruct((B, FF2, DP), F32),
    jax.ShapeDtypeStruct((B, 1, DP), F32),
    jax.ShapeDtypeStruct((B, E), F32),
)


def _routing0(logits0, W1s, b1s, W2s, b2s):
    return pl.pallas_call(_routing0_body, out_shape=_ROUT_OUT)(
        logits0, W1s, b1s, W2s, b2s)


def _routing(gi, gw, W1s, b1s, W2s, b2s):
    return pl.pallas_call(_routing_body, out_shape=_ROUT_OUT)(
        gi, gw, W1s, b1s, W2s, b2s)


def _ffn_y(Xb, w1_ref, b1_ref, w2_ref, b2_ref):
    h = jnp.dot(Xb, w1_ref[0], preferred_element_type=F32) + b1_ref[0]
    h = jnp.maximum(h, 0.0).astype(BF16)
    return jnp.dot(h, w2_ref[0], preferred_element_type=F32) + b2_ref[0]


def _node_sums(o):
    o3 = o.reshape(NT, L2, DP)
    return jnp.sum(jnp.sum(o3, axis=1), axis=1, keepdims=True)


def _layer0_body(x_ref, m_ref, s_ref, sw_ref, sb_ref,
                 w1_ref, b1_ref, w2_ref, b2_ref, out_ref, gate_ref):
    xn = (x_ref[0] - m_ref[0]) / s_ref[0]
    xe = xn[:, :L2]
    xo = xn[:, L2:]
    sw = sw_ref[...][None]
    sb = sb_ref[...][None]
    Xp = jnp.concatenate([xe[:, :, None] * sw + sb,
                          xo[:, :, None] * sw + sb], axis=2)
    X = Xp.reshape(RT2, DP)
    y = _ffn_y(X.astype(BF16), w1_ref, b1_ref, w2_ref, b2_ref)
    o = X + y
    out_ref[0] = o.astype(BF16)
    gate_ref[0, 0] = _node_sums(o)


def _layer_body(a_ref, gin_ref, w1_ref, b1_ref, w2_ref, b2_ref,
                out_ref, gate_ref):
    Xb = a_ref[0]
    y = _ffn_y(Xb, w1_ref, b1_ref, w2_ref, b2_ref)
    out_ref[0] = (Xb.astype(F32) + y).astype(BF16)
    gate_ref[0, 0] = gin_ref[0, 0] + _node_sums(y)


def _layer2_proj_body(a_ref, w1_ref, b1_ref, w2_ref, b2_ref,
                      pw_ref, pb_ref, m_ref, s_ref, o_ref):
    Xb = a_ref[0]
    y = _ffn_y(Xb, w1_ref, b1_ref, w2_ref, b2_ref)
    o3 = (Xb.astype(F32) + y).astype(BF16).reshape(NT, L2, DP)
    om = jnp.concatenate([o3[:, j, :] for j in range(L2)], axis=1)
    yp = jnp.dot(om, pw_ref[...], preferred_element_type=F32) + pb_ref[...]
    o_ref[0] = yp * s_ref[0] + m_ref[0]


_W_SPECS = [
    pl.BlockSpec((1, DP, FF2), lambda b, t: (b, 0, 0)),
    pl.BlockSpec((1, 1, FF2), lambda b, t: (b, 0, 0)),
    pl.BlockSpec((1, FF2, DP), lambda b, t: (b, 0, 0)),
    pl.BlockSpec((1, 1, DP), lambda b, t: (b, 0, 0)),
]
_A_SPEC = pl.BlockSpec((1, RT2, DP), lambda b, t: (b, t, 0))
_GATE_OUT_BF16 = (
    jax.ShapeDtypeStruct((B, NP * L2, DP), BF16),
    jax.ShapeDtypeStruct((B, TGRID, NT, 1), F32),
)
_GATE_OUT_SPECS = (
    _A_SPEC,
    pl.BlockSpec((1, 1, NT, 1), lambda b, t: (b, t, 0, 0)),
)


def _layer0(x_tp, m3, s3, sw, sb, w1c, b1c, w2c, b2c):
    return pl.pallas_call(
        _layer0_body,
        grid=(B, TGRID),
        in_specs=[
            pl.BlockSpec((1, NT, L), lambda b, t: (b, t, 0)),
            pl.BlockSpec((1, NT, 1), lambda b, t: (b, t, 0)),
            pl.BlockSpec((1, NT, 1), lambda b, t: (b, t, 0)),
            pl.BlockSpec((1, D), lambda b, t: (0, 0)),
            pl.BlockSpec((1, D), lambda b, t: (0, 0)),
            *_W_SPECS,
        ],
        out_specs=_GATE_OUT_SPECS,
        out_shape=_GATE_OUT_BF16,
    )(x_tp, m3, s3, sw, sb, w1c.astype(BF16), b1c, w2c.astype(BF16), b2c)


def _layer(A, gsum, w1c, b1c, w2c, b2c):
    return pl.pallas_call(
        _layer_body,
        grid=(B, TGRID),
        in_specs=[_A_SPEC,
                  pl.BlockSpec((1, 1, NT, 1), lambda b, t: (b, t, 0, 0)),
                  *_W_SPECS],
        out_specs=_GATE_OUT_SPECS,
        out_shape=_GATE_OUT_BF16,
    )(A, gsum, w1c.astype(BF16), b1c, w2c.astype(BF16), b2c)


def _layer2_proj(A, w1c, b1c, w2c, b2c, pw, pb, m3, s3):
    return pl.pallas_call(
        _layer2_proj_body,
        grid=(B, TGRID),
        in_specs=[
            _A_SPEC, *_W_SPECS,
            pl.BlockSpec((LD, P), lambda b, t: (0, 0)),
            pl.BlockSpec((1, P), lambda b, t: (0, 0)),
            pl.BlockSpec((1, NT, 1), lambda b, t: (b, t, 0)),
            pl.BlockSpec((1, NT, 1), lambda b, t: (b, t, 0)),
        ],
        out_specs=pl.BlockSpec((1, NT, P), lambda b, t: (b, t, 0)),
        out_shape=jax.ShapeDtypeStruct((B, NP, P), F32),
    )(A, w1c.astype(BF16), b1c, w2c.astype(BF16), b2c, pw, pb, m3, s3)


def _stats_body(g_ref, bal_ref, con_ref):
    g = g_ref[...]
    imp = jnp.sum(g, axis=1)
    mean = jnp.mean(imp, axis=1, keepdims=True)
    var = jnp.mean((imp - mean) ** 2, axis=1, keepdims=True)
    bal = var / (mean ** 2 + 1e-10)
    bal_ref[...] = jnp.sum(bal, axis=0, keepdims=True)
    con_l = -jnp.mean(jnp.sum(g * jnp.log(g + 1e-9), axis=2),
                      axis=1, keepdims=True)
    con_ref[...] = jnp.mean(con_l, axis=0, keepdims=True)


def _stats(gates_all):
    return pl.pallas_call(
        _stats_body,
        out_shape=(jax.ShapeDtypeStruct((1, 1), F32),
                   jax.ShapeDtypeStruct((1, 1), F32)),
    )(gates_all)


def kernel(x, start_w, start_b, gate_w, W1, b1, W2, b2, proj_w, proj_b):
    means = x.mean(axis=1, keepdims=True)
    std = jnp.sqrt(x.var(axis=1, keepdims=True) + 1e-5)
    xn = (x - means) / std
    out0 = xn[..., None] * start_w + start_b
    gate_in0 = out0.mean(axis=(1, 3))
    logits0 = gate_in0 @ gate_w[0]

    m3 = jnp.pad(means[:, 0, :], ((0, 0), (0, NP - N)))[:, :, None]
    s3 = jnp.pad(std[:, 0, :], ((0, 0), (0, NP - N)),
                 constant_values=1.0)[:, :, None]
    x_t = jnp.pad(x.transpose(0, 2, 1), ((0, 0), (0, NP - N), (0, 0)))
    x_tp = jnp.concatenate([x_t[:, :, 0::2], x_t[:, :, 1::2]], axis=2)
    gw_p = jnp.pad(gate_w, ((0, 0), (0, NP - N), (0, 0)))
    sw = start_w[None, :]
    sb = start_b[None, :]
    pb = proj_b[None, :]

    w1c, b1c, w2c, b2c, g0 = _routing0(logits0, W1[0], b1[0], W2[0], b2[0])
    A, gsum = _layer0(x_tp, m3, s3, sw, sb, w1c, b1c, w2c, b2c)
    w1c, b1c, w2c, b2c, g1 = _routing(gsum.reshape(B, NP), gw_p[1],
                                      W1[1], b1[1], W2[1], b2[1])
    A, gsum = _layer(A, gsum, w1c, b1c, w2c, b2c)
    w1c, b1c, w2c, b2c, g2 = _routing(gsum.reshape(B, NP), gw_p[2],
                                      W1[2], b1[2], W2[2], b2[2])
    o_nd = _layer2_proj(A, w1c, b1c, w2c, b2c, proj_w.astype(BF16),
                        pb, m3, s3)
    out = o_nd[:, :N, :].transpose(0, 2, 1)
    bal, con = _stats(jnp.stack([g0, g1, g2]))
    return out, bal[0, 0], con[0, 0]


# speedup vs baseline: 1.1129x; 1.0298x over previous
import jax
import jax.numpy as jnp
from jax.experimental import pallas as pl
from jax.experimental.pallas import tpu as pltpu

LAYERS = 3
N = 321
NP = 336
NT = 168
TGRID = NP // NT
L = 96
L2 = L // 2
D = 64
DP = 2 * D
FF = 64
FF2 = 4 * FF
E = 4
B = 8
P = 96
LD = L * D
RT2 = NT * L2
F32 = jnp.float32
BF16 = jnp.bfloat16


def _routing_compute(logits, W1s, b1s, W2s, b2s):
    col = jax.lax.broadcasted_iota(jnp.int32, (B, E), 1)
    m1 = jnp.max(logits, axis=1, keepdims=True)
    i1 = jnp.min(jnp.where(logits == m1, col, E), axis=1, keepdims=True)
    masked = jnp.where(col == i1, -jnp.inf, logits)
    m2 = jnp.max(masked, axis=1, keepdims=True)
    i2 = jnp.min(jnp.where(masked == m2, col, E), axis=1, keepdims=True)
    e2 = jnp.exp(m2 - m1)
    denom = 1.0 + e2
    g1 = 1.0 / denom
    g2 = e2 / denom
    gates = jnp.where(col == i1, g1, 0.0) + jnp.where(col == i2, g2, 0.0)

    w1a = jnp.zeros((B, D, FF), F32)
    w1b = jnp.zeros((B, D, FF), F32)
    w2a = jnp.zeros((B, FF, D), F32)
    w2b = jnp.zeros((B, FF, D), F32)
    b1a = jnp.zeros((B, FF), F32)
    b1b = jnp.zeros((B, FF), F32)
    b2c = jnp.zeros((B, D), F32)
    for e in range(E):
        s1 = (i1 == e).astype(F32)
        s2 = (i2 == e).astype(F32)
        sg1 = g1 * s1
        sg2 = g2 * s2
        w1a = w1a + s1[:, :, None] * W1s[e][None]
        w1b = w1b + s2[:, :, None] * W1s[e][None]
        w2a = w2a + sg1[:, :, None] * W2s[e][None]
        w2b = w2b + sg2[:, :, None] * W2s[e][None]
        b1a = b1a + s1 * b1s[e][None, :]
        b1b = b1b + s2 * b1s[e][None, :]
        b2c = b2c + (sg1 + sg2) * b2s[e][None, :]
    w1cat = jnp.concatenate([w1a, w1b], axis=2)
    w2cat = jnp.concatenate([w2a, w2b], axis=1)
    b1cat = jnp.concatenate([b1a, b1b], axis=1)
    zw1 = jnp.zeros((B, D, 2 * FF), F32)
    w1p = jnp.concatenate([
        jnp.concatenate([w1cat, zw1], axis=2),
        jnp.concatenate([zw1, w1cat], axis=2)], axis=1)
    zw2 = jnp.zeros((B, 2 * FF, D), F32)
    w2p = jnp.concatenate([
        jnp.concatenate([w2cat, zw2], axis=2),
        jnp.concatenate([zw2, w2cat], axis=2)], axis=1)
    b1p = jnp.concatenate([b1cat, b1cat], axis=1)
    b2p = jnp.concatenate([b2c, b2c], axis=1)
    return w1p, b1p[:, None, :], w2p, b2p[:, None, :], gates


def _routing0_body(lg_ref, W1_ref, b1_ref, W2_ref, b2_ref,
                   w1_ref, bb1_ref, w2_ref, bb2_ref, g_ref):
    o = _routing_compute(lg_ref[...], W1_ref, b1_ref, W2_ref, b2_ref)
    w1_ref[...], bb1_ref[...], w2_ref[...], bb2_ref[...], g_ref[...] = o


def _routing_body(gi_ref, gw_ref, W1_ref, b1_ref, W2_ref, b2_ref,
                  w1_ref, bb1_ref, w2_ref, bb2_ref, g_ref):
    logits = jnp.dot(gi_ref[...], gw_ref[...],
                     preferred_element_type=F32) * (1.0 / (L * D))
    o = _routing_compute(logits, W1_ref, b1_ref, W2_ref, b2_ref)
    w1_ref[...], bb1_ref[...], w2_ref[...], bb2_ref[...], g_ref[...] = o


_ROUT_OUT = (
    jax.ShapeDtypeStruct((B, DP, FF2), F32),
    jax.ShapeDtypeStruct((B, 1, FF2), F32),
    jax.ShapeDtypeStruct((B, FF2, DP), F32),
    jax.ShapeDtypeStruct((B, 1, DP), F32),
    jax.ShapeDtypeStruct((B, E), F32),
)


def _routing0(logits0, W1s, b1s, W2s, b2s):
    return pl.pallas_call(_routing0_body, out_shape=_ROUT_OUT)(
        logits0, W1s, b1s, W2s, b2s)


def _routing(gi, gw, W1s, b1s, W2s, b2s):
    return pl.pallas_call(_routing_body, out_shape=_ROUT_OUT)(
        gi, gw, W1s, b1s, W2s, b2s)


def _ffn_y(Xb, w1_ref, b1_ref, w2_ref, b2_ref):
    h = jnp.dot(Xb, w1_ref[0], preferred_element_type=F32) + b1_ref[0]
    h = jnp.maximum(h, 0.0).astype(BF16)
    return jnp.dot(h, w2_ref[0], preferred_element_type=F32) + b2_ref[0]


def _node_sums(o):
    o3 = o.reshape(NT, L2, DP)
    return jnp.sum(jnp.sum(o3, axis=1), axis=1, keepdims=True)


def _layer0_body(x_ref, m_ref, s_ref, sw_ref, sb_ref,
                 w1_ref, b1_ref, w2_ref, b2_ref, out_ref, gate_ref):
    xn = (x_ref[0] - m_ref[0]) / s_ref[0]
    xe = xn[:, :L2]
    xo = xn[:, L2:]
    sw = sw_ref[...][None]
    sb = sb_ref[...][None]
    Xp = jnp.concatenate([xe[:, :, None] * sw + sb,
                          xo[:, :, None] * sw + sb], axis=2)
    X = Xp.reshape(RT2, DP)
    y = _ffn_y(X.astype(BF16), w1_ref, b1_ref, w2_ref, b2_ref)
    o = X + y
    out_ref[0] = o.astype(BF16)
    gate_ref[0, 0] = _node_sums(o)


def _layer_body(a_ref, gin_ref, w1_ref, b1_ref, w2_ref, b2_ref,
                out_ref, gate_ref):
    Xb = a_ref[0]
    y = _ffn_y(Xb, w1_ref, b1_ref, w2_ref, b2_ref)
    out_ref[0] = (Xb.astype(F32) + y).astype(BF16)
    gate_ref[0, 0] = gin_ref[0, 0] + _node_sums(y)


def _layer2_proj_body(a_ref, w1_ref, b1_ref, w2_ref, b2_ref,
                      pw_ref, pb_ref, m_ref, s_ref, o_ref):
    Xb = a_ref[0]
    y = _ffn_y(Xb, w1_ref, b1_ref, w2_ref, b2_ref)
    o3 = (Xb.astype(F32) + y).astype(BF16).reshape(NT, L2, DP)
    om = jnp.concatenate([o3[:, j, :] for j in range(L2)], axis=1)
    yp = jnp.dot(om, pw_ref[...], preferred_element_type=F32) + pb_ref[...]
    o_ref[0] = yp * s_ref[0] + m_ref[0]


_W_SPECS = [
    pl.BlockSpec((1, DP, FF2), lambda b, t: (b, 0, 0)),
    pl.BlockSpec((1, 1, FF2), lambda b, t: (b, 0, 0)),
    pl.BlockSpec((1, FF2, DP), lambda b, t: (b, 0, 0)),
    pl.BlockSpec((1, 1, DP), lambda b, t: (b, 0, 0)),
]
_A_SPEC = pl.BlockSpec((1, RT2, DP), lambda b, t: (b, t, 0))
_GATE_OUT_BF16 = (
    jax.ShapeDtypeStruct((B, NP * L2, DP), BF16),
    jax.ShapeDtypeStruct((B, TGRID, NT, 1), F32),
)
_GATE_OUT_SPECS = (
    _A_SPEC,
    pl.BlockSpec((1, 1, NT, 1), lambda b, t: (b, t, 0, 0)),
)


def _layer0(x_tp, m3, s3, sw, sb, w1c, b1c, w2c, b2c):
    return pl.pallas_call(
        _layer0_body,
        grid=(B, TGRID),
        in_specs=[
            pl.BlockSpec((1, NT, L), lambda b, t: (b, t, 0)),
            pl.BlockSpec((1, NT, 1), lambda b, t: (b, t, 0)),
            pl.BlockSpec((1, NT, 1), lambda b, t: (b, t, 0)),
            pl.BlockSpec((1, D), lambda b, t: (0, 0)),
            pl.BlockSpec((1, D), lambda b, t: (0, 0)),
            *_W_SPECS,
        ],
        out_specs=_GATE_OUT_SPECS,
        out_shape=_GATE_OUT_BF16,
    )(x_tp, m3, s3, sw, sb, w1c.astype(BF16), b1c, w2c.astype(BF16), b2c)


def _layer(A, gsum, w1c, b1c, w2c, b2c):
    return pl.pallas_call(
        _layer_body,
        grid=(B, TGRID),
        in_specs=[_A_SPEC,
                  pl.BlockSpec((1, 1, NT, 1), lambda b, t: (b, t, 0, 0)),
                  *_W_SPECS],
        out_specs=_GATE_OUT_SPECS,
        out_shape=_GATE_OUT_BF16,
    )(A, gsum, w1c.astype(BF16), b1c, w2c.astype(BF16), b2c)


def _layer2_proj(A, w1c, b1c, w2c, b2c, pw, pb, m3, s3):
    return pl.pallas_call(
        _layer2_proj_body,
        grid=(B, TGRID),
        in_specs=[
            _A_SPEC, *_W_SPECS,
            pl.BlockSpec((LD, P), lambda b, t: (0, 0)),
            pl.BlockSpec((1, P), lambda b, t: (0, 0)),
            pl.BlockSpec((1, NT, 1), lambda b, t: (b, t, 0)),
            pl.BlockSpec((1, NT, 1), lambda b, t: (b, t, 0)),
        ],
        out_specs=pl.BlockSpec((1, NT, P), lambda b, t: (b, t, 0)),
        out_shape=jax.ShapeDtypeStruct((B, NP, P), F32),
    )(A, w1c.astype(BF16), b1c, w2c.astype(BF16), b2c, pw, pb, m3, s3)


def _stats_body(g_ref, bal_ref, con_ref):
    g = g_ref[...]
    imp = jnp.sum(g, axis=1)
    mean = jnp.mean(imp, axis=1, keepdims=True)
    var = jnp.mean((imp - mean) ** 2, axis=1, keepdims=True)
    bal = var / (mean ** 2 + 1e-10)
    bal_ref[...] = jnp.sum(bal, axis=0, keepdims=True)
    con_l = -jnp.mean(jnp.sum(g * jnp.log(g + 1e-9), axis=2),
                      axis=1, keepdims=True)
    con_ref[...] = jnp.mean(con_l, axis=0, keepdims=True)


def _stats(gates_all):
    return pl.pallas_call(
        _stats_body,
        out_shape=(jax.ShapeDtypeStruct((1, 1), F32),
                   jax.ShapeDtypeStruct((1, 1), F32)),
    )(gates_all)


def kernel(x, start_w, start_b, gate_w, W1, b1, W2, b2, proj_w, proj_b):
    means = x.mean(axis=1, keepdims=True)
    std = jnp.sqrt(x.var(axis=1, keepdims=True) + 1e-5)
    xn = (x - means) / std
    out0 = xn[..., None] * start_w + start_b
    gate_in0 = out0.mean(axis=(1, 3))
    logits0 = gate_in0 @ gate_w[0]

    m3 = jnp.pad(means[:, 0, :], ((0, 0), (0, NP - N)))[:, :, None]
    s3 = jnp.pad(std[:, 0, :], ((0, 0), (0, NP - N)),
                 constant_values=1.0)[:, :, None]
    x_t = jnp.pad(x.transpose(0, 2, 1), ((0, 0), (0, NP - N), (0, 0)))
    x_tp = jnp.concatenate([x_t[:, :, 0::2], x_t[:, :, 1::2]], axis=2)
    gw_p = jnp.pad(gate_w, ((0, 0), (0, NP - N), (0, 0)))
    sw = start_w[None, :]
    sb = start_b[None, :]
    pb = proj_b[None, :]

    w1c, b1c, w2c, b2c, g0 = _routing0(logits0, W1[0], b1[0], W2[0], b2[0])
    A, gsum = _layer0(x_tp, m3, s3, sw, sb, w1c, b1c, w2c, b2c)
    w1c, b1c, w2c, b2c, g1 = _routing(gsum.reshape(B, NP), gw_p[1],
                                      W1[1], b1[1], W2[1], b2[1])
    A, gsum = _layer(A, gsum, w1c, b1c, w2c, b2c)
    w1c, b1c, w2c, b2c, g2 = _routing(gsum.reshape(B, NP), gw_p[2],
                                      W1[2], b1[2], W2[2], b2[2])
    o_nd = _layer2_proj(A, w1c, b1c, w2c, b2c, proj_w.astype(BF16),
                        pb, m3, s3)
    out = o_nd[:, :N, :].transpose(0, 2, 1)
    bal, con = _stats(jnp.stack([g0, g1, g2]))
    return out, bal[0, 0], con[0, 0]


# speedup vs baseline: 1.1228x; 1.0088x over previous
import jax
import jax.numpy as jnp
from jax.experimental import pallas as pl
from jax.experimental.pallas import tpu as pltpu

LAYERS = 3
N = 321
NP = 336
NT = 336
TGRID = NP // NT
L = 96
L2 = L // 2
D = 64
DP = 2 * D
FF = 64
FF2 = 4 * FF
E = 4
B = 8
P = 96
LD = L * D
RT2 = NT * L2
F32 = jnp.float32
BF16 = jnp.bfloat16


def _routing_compute(logits, W1s, b1s, W2s, b2s):
    col = jax.lax.broadcasted_iota(jnp.int32, (B, E), 1)
    m1 = jnp.max(logits, axis=1, keepdims=True)
    i1 = jnp.min(jnp.where(logits == m1, col, E), axis=1, keepdims=True)
    masked = jnp.where(col == i1, -jnp.inf, logits)
    m2 = jnp.max(masked, axis=1, keepdims=True)
    i2 = jnp.min(jnp.where(masked == m2, col, E), axis=1, keepdims=True)
    e2 = jnp.exp(m2 - m1)
    denom = 1.0 + e2
    g1 = 1.0 / denom
    g2 = e2 / denom
    gates = jnp.where(col == i1, g1, 0.0) + jnp.where(col == i2, g2, 0.0)

    w1a = jnp.zeros((B, D, FF), F32)
    w1b = jnp.zeros((B, D, FF), F32)
    w2a = jnp.zeros((B, FF, D), F32)
    w2b = jnp.zeros((B, FF, D), F32)
    b1a = jnp.zeros((B, FF), F32)
    b1b = jnp.zeros((B, FF), F32)
    b2c = jnp.zeros((B, D), F32)
    for e in range(E):
        s1 = (i1 == e).astype(F32)
        s2 = (i2 == e).astype(F32)
        sg1 = g1 * s1
        sg2 = g2 * s2
        w1a = w1a + s1[:, :, None] * W1s[e][None]
        w1b = w1b + s2[:, :, None] * W1s[e][None]
        w2a = w2a + sg1[:, :, None] * W2s[e][None]
        w2b = w2b + sg2[:, :, None] * W2s[e][None]
        b1a = b1a + s1 * b1s[e][None, :]
        b1b = b1b + s2 * b1s[e][None, :]
        b2c = b2c + (sg1 + sg2) * b2s[e][None, :]
    w1cat = jnp.concatenate([w1a, w1b], axis=2)
    w2cat = jnp.concatenate([w2a, w2b], axis=1)
    b1cat = jnp.concatenate([b1a, b1b], axis=1)
    zw1 = jnp.zeros((B, D, 2 * FF), F32)
    w1p = jnp.concatenate([
        jnp.concatenate([w1cat, zw1], axis=2),
        jnp.concatenate([zw1, w1cat], axis=2)], axis=1)
    zw2 = jnp.zeros((B, 2 * FF, D), F32)
    w2p = jnp.concatenate([
        jnp.concatenate([w2cat, zw2], axis=2),
        jnp.concatenate([zw2, w2cat], axis=2)], axis=1)
    b1p = jnp.concatenate([b1cat, b1cat], axis=1)
    b2p = jnp.concatenate([b2c, b2c], axis=1)
    return w1p, b1p[:, None, :], w2p, b2p[:, None, :], gates


def _routing0_body(lg_ref, W1_ref, b1_ref, W2_ref, b2_ref,
                   w1_ref, bb1_ref, w2_ref, bb2_ref, g_ref):
    o = _routing_compute(lg_ref[...], W1_ref, b1_ref, W2_ref, b2_ref)
    w1_ref[...], bb1_ref[...], w2_ref[...], bb2_ref[...], g_ref[...] = o


def _routing_body(gi_ref, gw_ref, W1_ref, b1_ref, W2_ref, b2_ref,
                  w1_ref, bb1_ref, w2_ref, bb2_ref, g_ref):
    logits = jnp.dot(gi_ref[...], gw_ref[...],
                     preferred_element_type=F32) * (1.0 / (L * D))
    o = _routing_compute(logits, W1_ref, b1_ref, W2_ref, b2_ref)
    w1_ref[...], bb1_ref[...], w2_ref[...], bb2_ref[...], g_ref[...] = o


_ROUT_OUT = (
    jax.ShapeDtypeStruct((B, DP, FF2), F32),
    jax.ShapeDtypeStruct((B, 1, FF2), F32),
    jax.ShapeDtypeStruct((B, FF2, DP), F32),
    jax.ShapeDtypeStruct((B, 1, DP), F32),
    jax.ShapeDtypeStruct((B, E), F32),
)


def _routing0(logits0, W1s, b1s, W2s, b2s):
    return pl.pallas_call(_routing0_body, out_shape=_ROUT_OUT)(
        logits0, W1s, b1s, W2s, b2s)


def _routing(gi, gw, W1s, b1s, W2s, b2s):
    return pl.pallas_call(_routing_body, out_shape=_ROUT_OUT)(
        gi, gw, W1s, b1s, W2s, b2s)


def _ffn_y(Xb, w1_ref, b1_ref, w2_ref, b2_ref):
    h = jnp.dot(Xb, w1_ref[0], preferred_element_type=F32) + b1_ref[0]
    h = jnp.maximum(h, 0.0).astype(BF16)
    return jnp.dot(h, w2_ref[0], preferred_element_type=F32) + b2_ref[0]


def _node_sums(o):
    o3 = o.reshape(NT, L2, DP)
    return jnp.sum(jnp.sum(o3, axis=1), axis=1, keepdims=True)


def _layer0_body(x_ref, m_ref, s_ref, sw_ref, sb_ref,
                 w1_ref, b1_ref, w2_ref, b2_ref, out_ref, gate_ref):
    xn = (x_ref[0] - m_ref[0]) / s_ref[0]
    xe = xn[:, :L2]
    xo = xn[:, L2:]
    sw = sw_ref[...][None]
    sb = sb_ref[...][None]
    Xp = jnp.concatenate([xe[:, :, None] * sw + sb,
                          xo[:, :, None] * sw + sb], axis=2)
    X = Xp.reshape(RT2, DP)
    y = _ffn_y(X.astype(BF16), w1_ref, b1_ref, w2_ref, b2_ref)
    o = X + y
    out_ref[0] = o.astype(BF16)
    gate_ref[0, 0] = _node_sums(o)


def _layer_body(a_ref, gin_ref, w1_ref, b1_ref, w2_ref, b2_ref,
                out_ref, gate_ref):
    Xb = a_ref[0]
    y = _ffn_y(Xb, w1_ref, b1_ref, w2_ref, b2_ref)
    out_ref[0] = (Xb.astype(F32) + y).astype(BF16)
    gate_ref[0, 0] = gin_ref[0, 0] + _node_sums(y)


def _layer2_proj_body(a_ref, w1_ref, b1_ref, w2_ref, b2_ref,
                      pw_ref, pb_ref, m_ref, s_ref, o_ref):
    Xb = a_ref[0]
    y = _ffn_y(Xb, w1_ref, b1_ref, w2_ref, b2_ref)
    o3 = (Xb.astype(F32) + y).astype(BF16).reshape(NT, L2, DP)
    om = jnp.concatenate([o3[:, j, :] for j in range(L2)], axis=1)
    yp = jnp.dot(om, pw_ref[...], preferred_element_type=F32) + pb_ref[...]
    o_ref[0] = yp * s_ref[0] + m_ref[0]


_W_SPECS = [
    pl.BlockSpec((1, DP, FF2), lambda b, t: (b, 0, 0)),
    pl.BlockSpec((1, 1, FF2), lambda b, t: (b, 0, 0)),
    pl.BlockSpec((1, FF2, DP), lambda b, t: (b, 0, 0)),
    pl.BlockSpec((1, 1, DP), lambda b, t: (b, 0, 0)),
]
_A_SPEC = pl.BlockSpec((1, RT2, DP), lambda b, t: (b, t, 0))
_GATE_OUT_BF16 = (
    jax.ShapeDtypeStruct((B, NP * L2, DP), BF16),
    jax.ShapeDtypeStruct((B, TGRID, NT, 1), F32),
)
_GATE_OUT_SPECS = (
    _A_SPEC,
    pl.BlockSpec((1, 1, NT, 1), lambda b, t: (b, t, 0, 0)),
)


def _layer0(x_tp, m3, s3, sw, sb, w1c, b1c, w2c, b2c):
    return pl.pallas_call(
        _layer0_body,
        grid=(B, TGRID),
        in_specs=[
            pl.BlockSpec((1, NT, L), lambda b, t: (b, t, 0)),
            pl.BlockSpec((1, NT, 1), lambda b, t: (b, t, 0)),
            pl.BlockSpec((1, NT, 1), lambda b, t: (b, t, 0)),
            pl.BlockSpec((1, D), lambda b, t: (0, 0)),
            pl.BlockSpec((1, D), lambda b, t: (0, 0)),
            *_W_SPECS,
        ],
        out_specs=_GATE_OUT_SPECS,
        out_shape=_GATE_OUT_BF16,
    )(x_tp, m3, s3, sw, sb, w1c.astype(BF16), b1c, w2c.astype(BF16), b2c)


def _layer(A, gsum, w1c, b1c, w2c, b2c):
    return pl.pallas_call(
        _layer_body,
        grid=(B, TGRID),
        in_specs=[_A_SPEC,
                  pl.BlockSpec((1, 1, NT, 1), lambda b, t: (b, t, 0, 0)),
                  *_W_SPECS],
        out_specs=_GATE_OUT_SPECS,
        out_shape=_GATE_OUT_BF16,
    )(A, gsum, w1c.astype(BF16), b1c, w2c.astype(BF16), b2c)


def _layer2_proj(A, w1c, b1c, w2c, b2c, pw, pb, m3, s3):
    return pl.pallas_call(
        _layer2_proj_body,
        grid=(B, TGRID),
        in_specs=[
            _A_SPEC, *_W_SPECS,
            pl.BlockSpec((LD, P), lambda b, t: (0, 0)),
            pl.BlockSpec((1, P), lambda b, t: (0, 0)),
            pl.BlockSpec((1, NT, 1), lambda b, t: (b, t, 0)),
            pl.BlockSpec((1, NT, 1), lambda b, t: (b, t, 0)),
        ],
        out_specs=pl.BlockSpec((1, NT, P), lambda b, t: (b, t, 0)),
        out_shape=jax.ShapeDtypeStruct((B, NP, P), F32),
    )(A, w1c.astype(BF16), b1c, w2c.astype(BF16), b2c, pw, pb, m3, s3)


def _stats_body(g_ref, bal_ref, con_ref):
    g = g_ref[...]
    imp = jnp.sum(g, axis=1)
    mean = jnp.mean(imp, axis=1, keepdims=True)
    var = jnp.mean((imp - mean) ** 2, axis=1, keepdims=True)
    bal = var / (mean ** 2 + 1e-10)
    bal_ref[...] = jnp.sum(bal, axis=0, keepdims=True)
    con_l = -jnp.mean(jnp.sum(g * jnp.log(g + 1e-9), axis=2),
                      axis=1, keepdims=True)
    con_ref[...] = jnp.mean(con_l, axis=0, keepdims=True)


def _stats(gates_all):
    return pl.pallas_call(
        _stats_body,
        out_shape=(jax.ShapeDtypeStruct((1, 1), F32),
                   jax.ShapeDtypeStruct((1, 1), F32)),
    )(gates_all)


def kernel(x, start_w, start_b, gate_w, W1, b1, W2, b2, proj_w, proj_b):
    means = x.mean(axis=1, keepdims=True)
    std = jnp.sqrt(x.var(axis=1, keepdims=True) + 1e-5)
    xn = (x - means) / std
    out0 = xn[..., None] * start_w + start_b
    gate_in0 = out0.mean(axis=(1, 3))
    logits0 = gate_in0 @ gate_w[0]

    m3 = jnp.pad(means[:, 0, :], ((0, 0), (0, NP - N)))[:, :, None]
    s3 = jnp.pad(std[:, 0, :], ((0, 0), (0, NP - N)),
                 constant_values=1.0)[:, :, None]
    x_t = jnp.pad(x.transpose(0, 2, 1), ((0, 0), (0, NP - N), (0, 0)))
    x_tp = jnp.concatenate([x_t[:, :, 0::2], x_t[:, :, 1::2]], axis=2)
    gw_p = jnp.pad(gate_w, ((0, 0), (0, NP - N), (0, 0)))
    sw = start_w[None, :]
    sb = start_b[None, :]
    pb = proj_b[None, :]

    w1c, b1c, w2c, b2c, g0 = _routing0(logits0, W1[0], b1[0], W2[0], b2[0])
    A, gsum = _layer0(x_tp, m3, s3, sw, sb, w1c, b1c, w2c, b2c)
    w1c, b1c, w2c, b2c, g1 = _routing(gsum.reshape(B, NP), gw_p[1],
                                      W1[1], b1[1], W2[1], b2[1])
    A, gsum = _layer(A, gsum, w1c, b1c, w2c, b2c)
    w1c, b1c, w2c, b2c, g2 = _routing(gsum.reshape(B, NP), gw_p[2],
                                      W1[2], b1[2], W2[2], b2[2])
    o_nd = _layer2_proj(A, w1c, b1c, w2c, b2c, proj_w.astype(BF16),
                        pb, m3, s3)
    out = o_nd[:, :N, :].transpose(0, 2, 1)
    bal, con = _stats(jnp.stack([g0, g1, g2]))
    return out, bal[0, 0], con[0, 0]
